# Initial kernel scaffold; baseline (speedup 1.0000x reference)
import jax
import jax.numpy as jnp
from jax.experimental import pallas as pl


def kernel(token_ids, token_table, pos_table):
    raise NotImplementedError("write your pallas kernel here")


# speedup vs baseline: 1.2936x; 1.2936x over previous
import jax
import jax.numpy as jnp
from jax import lax
from jax.experimental import pallas as pl
from jax.experimental.pallas import tpu as pltpu
from jax.experimental.pallas import tpu_sc as plsc

BATCH = 4
SEQ = 4096
EMBED = 128
LANES = 16
NC, NS = 2, 16
NW = NC * NS
S_PER_W = SEQ // NW


def _emb_body(ids_hbm, tok_tab_hbm, pos_tab_hbm, out_hbm,
              idx_v, pos_v, tok_v, sem):
    wid = lax.axis_index("s") * NC + lax.axis_index("c")
    s0 = wid * S_PER_W
    for b in range(BATCH):
        pltpu.sync_copy(ids_hbm.at[b, pl.ds(s0, S_PER_W)], idx_v.at[b])
    pltpu.sync_copy(pos_tab_hbm.at[pl.ds(s0, S_PER_W)], pos_v)
    copies = [
        pltpu.async_copy(tok_tab_hbm.at[idx_v.at[b]], tok_v.at[b], sem)
        for b in range(BATCH)
    ]
    for c in copies:
        c.wait()

    def body(i, carry):
        for j in range(EMBED // LANES):
            sl = pl.ds(j * LANES, LANES)
            p = pos_v[i, sl]
            for b in range(BATCH):
                tok_v[b, i, sl] += p
        return carry

    lax.fori_loop(0, S_PER_W, body, 0)

    for b in range(BATCH):
        pltpu.sync_copy(tok_v.at[b], out_hbm.at[b, pl.ds(s0, S_PER_W)])


def kernel(token_ids, token_table, pos_table):
    token_ids = token_ids.astype(jnp.int32)
    f = pl.kernel(
        _emb_body,
        mesh=plsc.VectorSubcoreMesh(core_axis_name="c", subcore_axis_name="s"),
        out_type=jax.ShapeDtypeStruct((BATCH, SEQ, EMBED), jnp.float32),
        scratch_types=[
            pltpu.VMEM((BATCH, S_PER_W), jnp.int32),
            pltpu.VMEM((S_PER_W, EMBED), jnp.float32),
            pltpu.VMEM((BATCH, S_PER_W, EMBED), jnp.float32),
            pltpu.SemaphoreType.DMA,
        ],
    )
    return f(token_ids, token_table, pos_table)


# speedup vs baseline: 1.4292x; 1.1049x over previous
import jax
import jax.numpy as jnp
from jax import lax
from jax.experimental import pallas as pl
from jax.experimental.pallas import tpu as pltpu
from jax.experimental.pallas import tpu_sc as plsc

BATCH = 4
SEQ = 4096
EMBED = 128
LANES = 16
NC, NS = 2, 16
NW = NC * NS
S_PER_W = SEQ // NW
NCHUNK = 4
CS = S_PER_W // NCHUNK


def _emb_body(ids_hbm, tok_tab_hbm, pos_tab_hbm, out_hbm,
              idx_v, pos_v, tok_v, gsems, psem, osem):
    wid = lax.axis_index("s") * NC + lax.axis_index("c")
    s0 = wid * S_PER_W
    idx_copies = [
        pltpu.async_copy(ids_hbm.at[b, pl.ds(s0, S_PER_W)], idx_v.at[b],
                         gsems.at[NCHUNK])
        for b in range(BATCH)
    ]
    pos_copy = pltpu.async_copy(pos_tab_hbm.at[pl.ds(s0, S_PER_W)], pos_v, psem)
    for c in idx_copies:
        c.wait()
    gathers = [
        [pltpu.async_copy(
            tok_tab_hbm.at[idx_v.at[b, pl.ds(k * CS, CS)]],
            tok_v.at[b, pl.ds(k * CS, CS)],
            gsems.at[k])
         for b in range(BATCH)]
        for k in range(NCHUNK)
    ]
    pos_copy.wait()

    out_copies = []
    for k in range(NCHUNK):
        for c in gathers[k]:
            c.wait()

        def body(i, carry):
            for j in range(EMBED // LANES):
                sl = pl.ds(j * LANES, LANES)
                p = pos_v[i, sl]
                for b in range(BATCH):
                    tok_v[b, i, sl] += p
            return carry

        lax.fori_loop(k * CS, (k + 1) * CS, body, 0)
        out_copies.extend(
            pltpu.async_copy(tok_v.at[b, pl.ds(k * CS, CS)],
                             out_hbm.at[b, pl.ds(s0 + k * CS, CS)], osem)
            for b in range(BATCH))
    for c in out_copies:
        c.wait()


def kernel(token_ids, token_table, pos_table):
    token_ids = token_ids.astype(jnp.int32)
    f = pl.kernel(
        _emb_body,
        mesh=plsc.VectorSubcoreMesh(core_axis_name="c", subcore_axis_name="s"),
        out_type=jax.ShapeDtypeStruct((BATCH, SEQ, EMBED), jnp.float32),
        scratch_types=[
            pltpu.VMEM((BATCH, S_PER_W), jnp.int32),
            pltpu.VMEM((S_PER_W, EMBED), jnp.float32),
            pltpu.VMEM((BATCH, S_PER_W, EMBED), jnp.float32),
            pltpu.SemaphoreType.DMA((NCHUNK + 1,)),
            pltpu.SemaphoreType.DMA,
            pltpu.SemaphoreType.DMA,
        ],
    )
    return f(token_ids, token_table, pos_table)
